# Initial kernel scaffold; baseline (speedup 1.0000x reference)
import jax
import jax.numpy as jnp
from jax.experimental import pallas as pl


def kernel(edge_index, embedding_weight):
    raise NotImplementedError("write your pallas kernel here")


# speedup vs baseline: 9.9773x; 9.9773x over previous
import functools

import jax
import jax.numpy as jnp
from jax import lax
from jax.experimental import pallas as pl
from jax.experimental.pallas import tpu as pltpu
from jax.experimental.pallas import tpu_sc as plsc

N = 50000
D = 64
NUM_LAYERS = 3
E = 800000

NC = 2
NS = 16
LANES = 16

HALF = N // NC
ACC_ROWS = 25088
DUMP = HALF
SLAB = ACC_ROWS // NS
TAIL = HALF - (NS - 1) * SLAB

CH = 384
K = CH // 128
NCHUNK = 131
EPT = NCHUNK * CH
E_PAD = EPT * NS

_mesh = plsc.VectorSubcoreMesh(
    core_axis_name="c", subcore_axis_name="s", num_cores=NC, num_subcores=NS)
_sc_params = pltpu.CompilerParams(use_tc_tiling_on_sc=False)


def _compute_lidx(didx, lidx, base):
    for k in range(K):
        for v in range(128 // LANES):
            d = didx[pl.ds(k * 128 + v * LANES, LANES)]
            loc = d - base
            ok = (loc >= 0) & (loc < HALF)
            lidx[k, pl.ds(v * LANES, LANES)] = jnp.where(
                ok, loc, DUMP + (d & 63))


def _deg_body(dst_hbm, deg_hbm, didx, lidx, ones_v, zrow, dacc, sem):
    cid = lax.axis_index("c")
    sid = lax.axis_index("s")
    base = cid * HALF

    ov = jnp.ones((LANES,), jnp.float32)
    for v in range(128 // LANES):
        ones_v[pl.ds(v * LANES, LANES)] = ov
    zv = jnp.zeros((LANES,), jnp.float32)

    def zbody(i, _):
        zrow[pl.ds(i * LANES, LANES)] = zv
        return 0

    lax.fori_loop(0, SLAB // LANES, zbody, 0, unroll=4)
    pltpu.sync_copy(zrow, dacc.at[pl.ds(sid * SLAB, SLAB)])
    plsc.subcore_barrier()

    ebase = sid * EPT

    def chunk(j, _):
        off = ebase + j * CH
        pltpu.sync_copy(dst_hbm.at[pl.ds(off, CH)], didx)
        _compute_lidx(didx, lidx, base)
        for k in range(K):
            pltpu.sync_copy(ones_v, dacc.at[lidx.at[k]], add=True)
        return 0

    lax.fori_loop(0, NCHUNK, chunk, 0)
    plsc.subcore_barrier()

    @pl.when(sid < NS - 1)
    def _():
        pltpu.sync_copy(dacc.at[pl.ds(sid * SLAB, SLAB)], zrow)
        pltpu.sync_copy(zrow, deg_hbm.at[pl.ds(base + sid * SLAB, SLAB)])

    @pl.when(sid == NS - 1)
    def _():
        pltpu.sync_copy(dacc.at[pl.ds((NS - 1) * SLAB, TAIL)],
                        zrow.at[pl.ds(0, TAIL)])
        pltpu.sync_copy(zrow.at[pl.ds(0, TAIL)],
                        deg_hbm.at[pl.ds(base + (NS - 1) * SLAB, TAIL)])


_deg_call = functools.partial(
    pl.kernel,
    out_type=jax.ShapeDtypeStruct((N,), jnp.float32),
    mesh=_mesh,
    compiler_params=_sc_params,
    scratch_types=[
        pltpu.VMEM((CH,), jnp.int32),
        pltpu.VMEM((K, 128), jnp.int32),
        pltpu.VMEM((128,), jnp.float32),
        pltpu.VMEM((SLAB,), jnp.float32),
        pltpu.VMEM_SHARED((ACC_ROWS,), jnp.float32),
        pltpu.SemaphoreType.DMA,
    ],
)(_deg_body)


def _prop_body(z_hbm, src_hbm, dst_hbm, out_hbm,
               sidx, didx, lidx, rows, acc, sem):
    cid = lax.axis_index("c")
    sid = lax.axis_index("s")
    base = cid * HALF

    zv = jnp.zeros((LANES,), jnp.float32)

    def zbody(i, _):
        for c in range(D // LANES):
            rows[i, pl.ds(c * LANES, LANES)] = zv
        return 0

    lax.fori_loop(0, CH, zbody, 0, unroll=4)
    for k in range(SLAB // CH):
        pltpu.sync_copy(rows, acc.at[pl.ds(sid * SLAB + k * CH, CH)])
    rem = SLAB - (SLAB // CH) * CH
    if rem:
        pltpu.sync_copy(rows.at[pl.ds(0, rem)],
                        acc.at[pl.ds(sid * SLAB + (SLAB // CH) * CH, rem)])
    plsc.subcore_barrier()

    ebase = sid * EPT

    def chunk(j, _):
        off = ebase + j * CH
        pltpu.sync_copy(src_hbm.at[pl.ds(off, CH)], sidx)
        pltpu.sync_copy(dst_hbm.at[pl.ds(off, CH)], didx)
        _compute_lidx(didx, lidx, base)
        cps = [pltpu.async_copy(z_hbm.at[sidx.at[pl.ds(k * 128, 128)]],
                                rows.at[pl.ds(k * 128, 128)], sem)
               for k in range(K)]
        for cp in cps:
            cp.wait()
        for k in range(K):
            pltpu.sync_copy(rows.at[pl.ds(k * 128, 128)],
                            acc.at[lidx.at[k]], add=True)
        return 0

    lax.fori_loop(0, NCHUNK, chunk, 0)
    plsc.subcore_barrier()

    def bounce(off, nrows):
        pltpu.sync_copy(acc.at[pl.ds(off, nrows)], rows.at[pl.ds(0, nrows)])
        pltpu.sync_copy(rows.at[pl.ds(0, nrows)],
                        out_hbm.at[pl.ds(base + off, nrows)])

    @pl.when(sid < NS - 1)
    def _():
        for k in range(SLAB // CH):
            bounce(sid * SLAB + k * CH, CH)
        if SLAB - (SLAB // CH) * CH:
            bounce(sid * SLAB + (SLAB // CH) * CH, SLAB - (SLAB // CH) * CH)

    @pl.when(sid == NS - 1)
    def _():
        for k in range(TAIL // CH):
            bounce((NS - 1) * SLAB + k * CH, CH)
        if TAIL - (TAIL // CH) * CH:
            bounce((NS - 1) * SLAB + (TAIL // CH) * CH,
                   TAIL - (TAIL // CH) * CH)


_prop_call = functools.partial(
    pl.kernel,
    out_type=jax.ShapeDtypeStruct((N, D), jnp.float32),
    mesh=_mesh,
    compiler_params=_sc_params,
    scratch_types=[
        pltpu.VMEM((CH,), jnp.int32),
        pltpu.VMEM((CH,), jnp.int32),
        pltpu.VMEM((K, 128), jnp.int32),
        pltpu.VMEM((CH, D), jnp.float32),
        pltpu.VMEM_SHARED((ACC_ROWS, D), jnp.float32),
        pltpu.SemaphoreType.DMA,
    ],
)(_prop_body)


BR = 5000


def _scale_body(deg_ref, x_ref, dis_ref, z_ref):
    deg = deg_ref[...]
    dis = jnp.where(deg > 0.0, lax.rsqrt(jnp.maximum(deg, 1e-12)), 0.0)
    dis_ref[...] = dis
    z_ref[...] = dis * x_ref[...]


def _scale_call(deg2, x):
    return pl.pallas_call(
        _scale_body,
        grid=(N // BR,),
        in_specs=[
            pl.BlockSpec((BR, 1), lambda i: (i, 0)),
            pl.BlockSpec((BR, D), lambda i: (i, 0)),
        ],
        out_specs=[
            pl.BlockSpec((BR, 1), lambda i: (i, 0)),
            pl.BlockSpec((BR, D), lambda i: (i, 0)),
        ],
        out_shape=[
            jax.ShapeDtypeStruct((N, 1), jnp.float32),
            jax.ShapeDtypeStruct((N, D), jnp.float32),
        ],
    )(deg2, x)


def _layer_body(s_ref, dis_ref, acc_ref, accout_ref, z_ref):
    dis = dis_ref[...]
    xp = dis * s_ref[...]
    accout_ref[...] = acc_ref[...] + xp
    z_ref[...] = dis * xp


def _layer_call(s, dis, acc):
    return pl.pallas_call(
        _layer_body,
        grid=(N // BR,),
        in_specs=[
            pl.BlockSpec((BR, D), lambda i: (i, 0)),
            pl.BlockSpec((BR, 1), lambda i: (i, 0)),
            pl.BlockSpec((BR, D), lambda i: (i, 0)),
        ],
        out_specs=[
            pl.BlockSpec((BR, D), lambda i: (i, 0)),
            pl.BlockSpec((BR, D), lambda i: (i, 0)),
        ],
        out_shape=[
            jax.ShapeDtypeStruct((N, D), jnp.float32),
            jax.ShapeDtypeStruct((N, D), jnp.float32),
        ],
    )(s, dis, acc)


def _last_body(s_ref, dis_ref, acc_ref, out_ref):
    xp = dis_ref[...] * s_ref[...]
    out_ref[...] = (acc_ref[...] + xp) * (1.0 / (NUM_LAYERS + 1))


def _last_call(s, dis, acc):
    return pl.pallas_call(
        _last_body,
        grid=(N // BR,),
        in_specs=[
            pl.BlockSpec((BR, D), lambda i: (i, 0)),
            pl.BlockSpec((BR, 1), lambda i: (i, 0)),
            pl.BlockSpec((BR, D), lambda i: (i, 0)),
        ],
        out_specs=pl.BlockSpec((BR, D), lambda i: (i, 0)),
        out_shape=jax.ShapeDtypeStruct((N, D), jnp.float32),
    )(s, dis, acc)


def kernel(edge_index, embedding_weight):
    src = edge_index[0]
    dst = edge_index[1]
    pad = E_PAD - E
    src_p = jnp.concatenate([src, jnp.zeros((pad,), jnp.int32)])
    dst_p = jnp.concatenate([dst, jnp.full((pad,), -1, jnp.int32)])

    deg = _deg_call(dst_p)
    dis, z = _scale_call(deg.reshape(N, 1), embedding_weight)

    acc = embedding_weight
    out = None
    for l in range(NUM_LAYERS):
        s = _prop_call(z, src_p, dst_p)
        if l < NUM_LAYERS - 1:
            acc, z = _layer_call(s, dis, acc)
        else:
            out = _last_call(s, dis, acc)
    return out
